# baseline (speedup 1.0000x reference)
import functools

import jax
import jax.numpy as jnp
from jax import lax
from jax.experimental import pallas as pl
from jax.experimental.pallas import tpu as pltpu
from jax.experimental.pallas import tpu_sc as plsc

_N_EDGES = 320000
_HIDDEN = 128
_HP = _HIDDEN // 2
_NC = 2
_NS = 16
_NW = _NC * _NS
_EPW = _N_EDGES // _NW
_C = 128
_NCHUNK = -(-_EPW // _C)
_EPAD = _NCHUNK * _C


def _sc_score(idx, z, rel):
    mesh = plsc.VectorSubcoreMesh(core_axis_name="c", subcore_axis_name="s")

    @functools.partial(
        pl.kernel,
        mesh=mesh,
        compiler_params=pltpu.CompilerParams(needs_layout_passes=False),
        out_type=jax.ShapeDtypeStruct((_N_EDGES,), jnp.float32),
        scratch_types=[
            pltpu.VMEM((2, 3, _C), jnp.int32),
            pltpu.VMEM((2, _C, _HIDDEN), jnp.float32),
            pltpu.VMEM((2, _C, _HIDDEN), jnp.float32),
            pltpu.VMEM((2, _C, _HIDDEN), jnp.float32),
            pltpu.VMEM((_EPAD,), jnp.float32),
            pltpu.SemaphoreType.DMA,
            pltpu.SemaphoreType.DMA,
            pltpu.SemaphoreType.DMA,
            pltpu.SemaphoreType.DMA,
        ],
    )
    def k(idx_hbm, z_hbm, rel_hbm, out_hbm,
          iv_v, sr_v, dr_v, rr_v, ob_v, si0, si1, sg0, sg1):
        sem_i = (si0, si1)
        sem_g = (sg0, sg1)
        wid = lax.axis_index("s") * _NC + lax.axis_index("c")
        base = wid * _EPW
        lane = lax.iota(jnp.int32, 16)

        def issue_idx(kk, b):
            pltpu.async_copy(idx_hbm.at[wid * _NCHUNK + kk], iv_v.at[b], sem_i[b])

        def wait_idx(b):
            pltpu.make_async_copy(
                idx_hbm.at[0], iv_v.at[b], sem_i[b]).wait()

        def issue_gather(b):
            pltpu.async_copy(z_hbm.at[iv_v.at[b, 0]], sr_v.at[b], sem_g[b])
            pltpu.async_copy(z_hbm.at[iv_v.at[b, 1]], dr_v.at[b], sem_g[b])
            pltpu.async_copy(rel_hbm.at[iv_v.at[b, 2]], rr_v.at[b], sem_g[b])

        def wait_gather(b):
            for buf in (sr_v, dr_v, rr_v):
                pltpu.make_async_copy(
                    z_hbm.at[pl.ds(0, _C)], buf.at[b], sem_g[b]).wait()

        issue_idx(0, 0)
        issue_idx(1, 1)
        wait_idx(0)
        issue_gather(0)

        def pair(kk2, carry):
            for b in (0, 1):
                kk = kk2 * 2 + b

                @pl.when(kk < _NCHUNK)
                def _():
                    wait_gather(b)

                    @pl.when(kk + 2 < _NCHUNK)
                    def _():
                        issue_idx(kk + 2, b)

                    @pl.when(kk + 1 < _NCHUNK)
                    def _():
                        wait_idx(1 - b)
                        issue_gather(1 - b)

                    obase = kk * _C

                    def group(g, c2):
                        res = jnp.zeros((16,), jnp.float32)
                        for el in range(16):
                            e = g * 16 + el
                            acc0 = jnp.zeros((16,), jnp.float32)
                            acc1 = jnp.zeros((16,), jnp.float32)
                            for j in range(_HIDDEN // 32):
                                sp = sr_v[b, e, pl.ds(j * 16, 16)]
                                dp = dr_v[b, e, pl.ds(j * 16, 16)]
                                rp = rr_v[b, e, pl.ds(j * 16, 16)]
                                p = (plsc.bitcast(sp, jnp.bfloat16)
                                     * plsc.bitcast(dp, jnp.bfloat16)
                                     * plsc.bitcast(rp, jnp.bfloat16))
                                p0, p1 = plsc.unpack(
                                    p, format=plsc.PackFormat.INTERLEAVED)
                                acc0 = acc0 + p0
                                acc1 = acc1 + p1
                            res = jnp.where(lane == el, jnp.sum(acc0 + acc1), res)
                        ob_v[pl.ds(obase + g * 16, 16)] = res
                        return c2

                    lax.fori_loop(0, _C // 16, group, 0)

            return carry

        lax.fori_loop(0, (_NCHUNK + 1) // 2, pair, 0)
        pltpu.sync_copy(ob_v.at[pl.ds(0, _EPW)], out_hbm.at[pl.ds(base, _EPW)])

    return k(idx, z, rel)


def _pack_bf16(t):
    tb = t.astype(jnp.bfloat16)
    td = jnp.concatenate([tb, tb], axis=1)
    return jax.lax.bitcast_convert_type(
        td.reshape(t.shape[0], _HIDDEN, 2), jnp.float32)


def kernel(z, edge_index, edge_type, rel_emb):
    ei = edge_index.astype(jnp.int32)
    ty = edge_type.astype(jnp.int32)
    idx = jnp.stack([ei[0], ei[1], ty], axis=0)
    idx = idx.reshape(3, _NW, _EPW)
    idx = jnp.pad(idx, ((0, 0), (0, 0), (0, _EPAD - _EPW)))
    idx = idx.reshape(3, _NW * _NCHUNK, _C).transpose(1, 0, 2)
    return _sc_score(idx, _pack_bf16(z), _pack_bf16(rel_emb))


# speedup vs baseline: 1.6331x; 1.6331x over previous
import functools

import jax
import jax.numpy as jnp
from jax import lax
from jax.experimental import pallas as pl
from jax.experimental.pallas import tpu as pltpu
from jax.experimental.pallas import tpu_sc as plsc

_N_EDGES = 320000
_HIDDEN = 128
_HP = _HIDDEN // 2
_NC = 2
_NS = 16
_NW = _NC * _NS
_EPW = _N_EDGES // _NW
_C = 80
_NCHUNK = _EPW // _C


def _sc_score(src, dst, typ, z, rel):
    mesh = plsc.VectorSubcoreMesh(core_axis_name="c", subcore_axis_name="s")

    @functools.partial(
        pl.kernel,
        mesh=mesh,
        compiler_params=pltpu.CompilerParams(needs_layout_passes=False),
        out_type=jax.ShapeDtypeStruct((_N_EDGES,), jnp.float32),
        scratch_types=[
            pltpu.VMEM((2, 3, _C), jnp.int32),
            pltpu.VMEM((2, _C, _HIDDEN), jnp.float32),
            pltpu.VMEM((2, _C, _HIDDEN), jnp.float32),
            pltpu.VMEM((2, _C, _HIDDEN), jnp.float32),
            pltpu.VMEM((_EPW,), jnp.float32),
            pltpu.SemaphoreType.DMA,
            pltpu.SemaphoreType.DMA,
            pltpu.SemaphoreType.DMA,
            pltpu.SemaphoreType.DMA,
        ],
    )
    def k(src_hbm, dst_hbm, typ_hbm, z_hbm, rel_hbm, out_hbm,
          iv_v, sr_v, dr_v, rr_v, ob_v, si0, si1, sg0, sg1):
        sem_i = (si0, si1)
        sem_g = (sg0, sg1)
        wid = lax.axis_index("s") * _NC + lax.axis_index("c")
        base = wid * _EPW
        lane = lax.iota(jnp.int32, 16)

        def issue_idx(kk, b):
            off = base + kk * _C
            pltpu.async_copy(src_hbm.at[pl.ds(off, _C)], iv_v.at[b, 0], sem_i[b])
            pltpu.async_copy(dst_hbm.at[pl.ds(off, _C)], iv_v.at[b, 1], sem_i[b])
            pltpu.async_copy(typ_hbm.at[pl.ds(off, _C)], iv_v.at[b, 2], sem_i[b])

        def wait_idx(b):
            for j in range(3):
                pltpu.make_async_copy(
                    src_hbm.at[pl.ds(0, _C)], iv_v.at[b, j], sem_i[b]).wait()

        def issue_gather(b):
            pltpu.async_copy(z_hbm.at[iv_v.at[b, 0]], sr_v.at[b], sem_g[b])
            pltpu.async_copy(z_hbm.at[iv_v.at[b, 1]], dr_v.at[b], sem_g[b])
            pltpu.async_copy(rel_hbm.at[iv_v.at[b, 2]], rr_v.at[b], sem_g[b])

        def wait_gather(b):
            for buf in (sr_v, dr_v, rr_v):
                pltpu.make_async_copy(
                    z_hbm.at[pl.ds(0, _C)], buf.at[b], sem_g[b]).wait()

        issue_idx(0, 0)
        issue_idx(1, 1)
        wait_idx(0)
        issue_gather(0)

        def pair(kk2, carry):
            for b in (0, 1):
                kk = kk2 * 2 + b

                @pl.when(kk < _NCHUNK)
                def _():
                    wait_gather(b)

                    @pl.when(kk + 2 < _NCHUNK)
                    def _():
                        issue_idx(kk + 2, b)

                    @pl.when(kk + 1 < _NCHUNK)
                    def _():
                        wait_idx(1 - b)
                        issue_gather(1 - b)

                    obase = kk * _C

                    def group(g, c2):
                        res = jnp.zeros((16,), jnp.float32)
                        for el in range(16):
                            e = g * 16 + el
                            acc0 = jnp.zeros((16,), jnp.float32)
                            acc1 = jnp.zeros((16,), jnp.float32)
                            for j in range(_HIDDEN // 32):
                                sp = sr_v[b, e, pl.ds(j * 16, 16)]
                                dp = dr_v[b, e, pl.ds(j * 16, 16)]
                                rp = rr_v[b, e, pl.ds(j * 16, 16)]
                                p = (plsc.bitcast(sp, jnp.bfloat16)
                                     * plsc.bitcast(dp, jnp.bfloat16)
                                     * plsc.bitcast(rp, jnp.bfloat16))
                                p0, p1 = plsc.unpack(
                                    p, format=plsc.PackFormat.INTERLEAVED)
                                acc0 = acc0 + p0
                                acc1 = acc1 + p1
                            res = jnp.where(lane == el, jnp.sum(acc0 + acc1), res)
                        ob_v[pl.ds(obase + g * 16, 16)] = res
                        return c2

                    lax.fori_loop(0, _C // 16, group, 0)

            return carry

        lax.fori_loop(0, (_NCHUNK + 1) // 2, pair, 0)
        pltpu.sync_copy(ob_v, out_hbm.at[pl.ds(base, _EPW)])

    return k(src, dst, typ, z, rel)


def _pack_bf16(t):
    tb = t.astype(jnp.bfloat16)
    td = jnp.concatenate([tb, tb], axis=1)
    return jax.lax.bitcast_convert_type(
        td.reshape(t.shape[0], _HIDDEN, 2), jnp.float32)


def kernel(z, edge_index, edge_type, rel_emb):
    ei = edge_index.astype(jnp.int32)
    return _sc_score(ei[0], ei[1], edge_type.astype(jnp.int32),
                     _pack_bf16(z), _pack_bf16(rel_emb))
